# baseline (speedup 1.0000x reference)
import functools

import jax
import jax.numpy as jnp
from jax import lax
from jax.experimental import pallas as pl
from jax.experimental.pallas import tpu as pltpu
from jax.experimental.pallas import tpu_sc as plsc

_L = 16


def _stitch_body(n_rows, n_cols, part_hbm, idx0_hbm, data_hbm, out_hbm,
                 part_v, idx0_v, data_v, nz_v, out_v):
    wid = lax.axis_index("s") * 2 + lax.axis_index("c")

    @pl.when(wid == 0)
    def _():
        pltpu.sync_copy(part_hbm, part_v)
        pltpu.sync_copy(idx0_hbm, idx0_v)
        pltpu.sync_copy(data_hbm, data_v)

        lanes = lax.iota(jnp.int32, _L)
        in_part0 = (part_v[...] == 0) & (lanes < n_rows)
        rank = plsc.cumsum(jnp.where(in_part0, 1, 0)) - 1
        nz_v[...] = jnp.zeros((_L,), jnp.int32)
        plsc.store_scatter(nz_v, [rank], lanes, mask=in_part0)

        rowid = lanes // n_cols
        colid = lanes - rowid * n_cols
        valid = lanes < n_rows * n_cols

        src_row = plsc.load_gather(nz_v, [rowid])
        part0 = plsc.load_gather(data_v, [src_row * n_cols + colid], mask=valid)

        dst_row = plsc.load_gather(idx0_v, [rowid])
        out_v[...] = jnp.zeros((_L,), jnp.float32)
        plsc.store_scatter(out_v, [dst_row * n_cols + colid], part0, mask=valid)

        pltpu.sync_copy(out_v, out_hbm)


@functools.partial(jax.jit, static_argnums=(4, 5))
def _stitch(part_p, idx0_p, data_p, _unused, n_rows, n_cols):
    body = functools.partial(_stitch_body, n_rows, n_cols)
    return pl.kernel(
        body,
        out_type=jax.ShapeDtypeStruct((_L,), jnp.float32),
        mesh=plsc.VectorSubcoreMesh(
            core_axis_name="c", subcore_axis_name="s",
            num_cores=2, num_subcores=16,
        ),
        scratch_types=[
            pltpu.VMEM((_L,), jnp.int32),
            pltpu.VMEM((_L,), jnp.int32),
            pltpu.VMEM((_L,), jnp.float32),
            pltpu.VMEM((_L,), jnp.int32),
            pltpu.VMEM((_L,), jnp.float32),
        ],
        compiler_params=pltpu.CompilerParams(needs_layout_passes=False),
    )(part_p, idx0_p, data_p)


def kernel(data, partitions, index0, index1):
    n_rows, n_cols = data.shape
    assert n_rows * n_cols <= _L and index0.shape[0] <= _L
    assert index1.shape[0] == 0

    part_p = jnp.pad(partitions, (0, _L - partitions.shape[0]))
    idx0_p = jnp.pad(index0, (0, _L - index0.shape[0]))
    data_p = jnp.pad(data.reshape(-1), (0, _L - n_rows * n_cols))

    out = _stitch(part_p, idx0_p, data_p, index1, n_rows, n_cols)
    return out[: n_rows * n_cols].reshape(n_rows, n_cols)


# speedup vs baseline: 1.0819x; 1.0819x over previous
import functools

import jax
import jax.numpy as jnp
from jax import lax
from jax.experimental import pallas as pl
from jax.experimental.pallas import tpu as pltpu
from jax.experimental.pallas import tpu_sc as plsc

_L = 16


def _stitch_body(n_rows, n_cols, m0, part_hbm, idx0_hbm, data_hbm, out_hbm,
                 part_v, idx0_v, data_v, nz_v, out_v):
    pltpu.sync_copy(part_hbm, part_v)
    pltpu.sync_copy(idx0_hbm, idx0_v)
    pltpu.sync_copy(data_hbm, data_v)

    lanes = lax.iota(jnp.int32, _L)
    zeros = jnp.zeros((_L,), jnp.float32)

    row = jnp.minimum(lanes // n_cols, m0 - 1)
    part = plsc.load_gather(part_v, [jnp.minimum(lanes, n_rows - 1)])
    in_part0 = (part == 0) & (lanes < n_rows)
    rank = plsc.cumsum(jnp.where(in_part0, 1, 0)) - 1
    plsc.store_scatter(nz_v, [jnp.minimum(lanes, m0 - 1)],
                       jnp.zeros((_L,), jnp.int32), mask=lanes < m0)
    plsc.store_scatter(nz_v, [rank], lanes, mask=in_part0)

    col = lanes - (lanes // n_cols) * n_cols
    valid = lanes < m0 * n_cols

    src_row = plsc.load_gather(nz_v, [row])
    part0 = plsc.load_gather(data_v, [src_row, col], mask=valid)

    dst_row = plsc.load_gather(idx0_v, [row])
    dst_ok = valid & (dst_row >= 0) & (dst_row < n_rows)
    dst_row = jnp.clip(dst_row, 0, n_rows - 1)
    plsc.store_scatter(out_v, [jnp.minimum(lanes, n_rows * n_cols - 1)],
                       zeros, mask=lanes < n_rows * n_cols)
    plsc.store_scatter(out_v, [dst_row * n_cols + col], part0, mask=dst_ok)

    pltpu.sync_copy(out_v, out_hbm)


def kernel(data, partitions, index0, index1):
    n_rows, n_cols = data.shape
    m0 = index0.shape[0]
    assert n_rows * n_cols <= _L and m0 * n_cols <= _L
    assert index1.shape[0] == 0

    body = functools.partial(_stitch_body, n_rows, n_cols, m0)
    out = pl.kernel(
        body,
        out_type=jax.ShapeDtypeStruct((n_rows * n_cols,), jnp.float32),
        mesh=plsc.VectorSubcoreMesh(
            core_axis_name="c", subcore_axis_name="s",
            num_cores=1, num_subcores=1,
        ),
        scratch_types=[
            pltpu.VMEM((n_rows,), jnp.int32),
            pltpu.VMEM((m0,), jnp.int32),
            pltpu.VMEM((n_rows, n_cols), jnp.float32),
            pltpu.VMEM((m0,), jnp.int32),
            pltpu.VMEM((n_rows * n_cols,), jnp.float32),
        ],
        compiler_params=pltpu.CompilerParams(needs_layout_passes=False),
    )(partitions, index0, data)
    return out.reshape(n_rows, n_cols)
